# baseline (speedup 1.0000x reference)
import jax
import jax.numpy as jnp
import numpy as np
from jax import lax
from jax.experimental import pallas as pl
from jax.experimental.pallas import tpu as pltpu
from jax.experimental.pallas import tpu_sc as plsc

B = 16384
NCAT = 26
DNUM = 13
DT = 32
NTOK = 1 + DNUM + NCAT
NC = 2
NS = 16
NW = NC * NS
BPW = B // NW
NB = 32
NCHUNK = BPW // NB
F = NB * NCAT
G = 64
NG = F // G
NUMROWS = NB * (DNUM + 1)
GN = NUMROWS // G
PERIOD = 208


def _body(xnum_hbm, xcat_hbm, w_hbm, b_hbm, table_hbm, offs_hbm, dstc_hbm,
          dstn_hbm, out_hbm,
          xcat_v, xnum_v, w_v, b_v, offs_v, dstc_v, dstn_v,
          idx_v, di_v, ni_v, temp_v, num_v, sem):
    cid = lax.axis_index("c")
    sid = lax.axis_index("s")
    wid = sid * NC + cid
    bb0 = wid * BPW
    fb0 = bb0 * NCAT

    pltpu.sync_copy(xcat_hbm.at[pl.ds(fb0, BPW * NCAT)], xcat_v)
    pltpu.sync_copy(xnum_hbm.at[:, pl.ds(bb0, BPW)], xnum_v)
    pltpu.sync_copy(w_hbm, w_v)
    pltpu.sync_copy(b_hbm, b_v)
    pltpu.sync_copy(offs_hbm, offs_v)
    pltpu.sync_copy(dstc_hbm, dstc_v)
    pltpu.sync_copy(dstn_hbm, dstn_v)

    def chunk(t, carry):
        f0 = t * F
        orow0 = (bb0 + t * NB) * NTOK

        for g in range(NG):
            for m in range(G // 16):
                o = g * G + m * 16
                sl = pl.ds(m * 16, 16)
                idx_v[g, sl] = (xcat_v[pl.ds(f0 + o, 16)]
                                + offs_v[pl.ds(o % PERIOD, 16)])
                di_v[g, sl] = dstc_v[pl.ds(o, 16)] + orow0
        for g in range(GN):
            for m in range(G // 16):
                o = g * G + m * 16
                ni_v[g, pl.ds(m * 16, 16)] = dstn_v[pl.ds(o, 16)] + orow0

        handles = [
            pltpu.async_copy(table_hbm.at[idx_v.at[g]],
                             temp_v.at[pl.ds(g * G, G)], sem)
            for g in range(NG)
        ]
        for h in handles:
            h.wait()

        def biasb(bb, c2):
            r = bb * NCAT
            for c in range(NCAT):
                for h2 in range(2):
                    sl = pl.ds(h2 * 16, 16)
                    temp_v[r + c, sl] = temp_v[r + c, sl] + b_v[13 + c, sl]
            return c2

        lax.fori_loop(0, NB, biasb, 0)

        for q in range(NB // 16):
            xvs = [xnum_v[j, pl.ds(t * NB + q * 16, 16)] for j in range(DNUM)]
            for l in range(16):
                r = (q * 16 + l) * (DNUM + 1)
                for h2 in range(2):
                    sl = pl.ds(h2 * 16, 16)
                    num_v[r, sl] = w_v[0, sl]
                for j in range(1, DNUM + 1):
                    xs = xvs[j - 1][l]
                    for h2 in range(2):
                        sl = pl.ds(h2 * 16, 16)
                        num_v[r + j, sl] = w_v[j, sl] * xs + b_v[j - 1, sl]

        hs = [
            pltpu.async_copy(temp_v.at[pl.ds(g * G, G)],
                             out_hbm.at[di_v.at[g]], sem)
            for g in range(NG)
        ]
        hs += [
            pltpu.async_copy(num_v.at[pl.ds(g * G, G)],
                             out_hbm.at[ni_v.at[g]], sem)
            for g in range(GN)
        ]
        for h in hs:
            h.wait()
        return carry

    lax.fori_loop(0, NCHUNK, chunk, 0)


def kernel(x_num, x_cat, weight, bias, emb_table, category_offsets):
    xcat_flat = x_cat.reshape(-1)
    offs_pat = jnp.tile(category_offsets, PERIOD // NCAT)

    tc = np.arange(F, dtype=np.int32)
    dstc = jnp.asarray((tc // NCAT) * NTOK + (1 + DNUM) + (tc % NCAT))
    tn = np.arange(NUMROWS, dtype=np.int32)
    dstn = jnp.asarray((tn // (DNUM + 1)) * NTOK + (tn % (DNUM + 1)))

    kfn = pl.kernel(
        _body,
        out_type=jax.ShapeDtypeStruct((B * NTOK, DT), jnp.float32),
        mesh=plsc.VectorSubcoreMesh(core_axis_name="c", subcore_axis_name="s"),
        compiler_params=pltpu.CompilerParams(use_tc_tiling_on_sc=False),
        scratch_types=[
            pltpu.VMEM((BPW * NCAT,), jnp.int32),
            pltpu.VMEM((DNUM, BPW), jnp.float32),
            pltpu.VMEM((DNUM + 1, DT), jnp.float32),
            pltpu.VMEM((DNUM + NCAT, DT), jnp.float32),
            pltpu.VMEM((PERIOD,), jnp.int32),
            pltpu.VMEM((F,), jnp.int32),
            pltpu.VMEM((NUMROWS,), jnp.int32),
            pltpu.VMEM((NG, G), jnp.int32),
            pltpu.VMEM((NG, G), jnp.int32),
            pltpu.VMEM((GN, G), jnp.int32),
            pltpu.VMEM((F, DT), jnp.float32),
            pltpu.VMEM((NUMROWS, DT), jnp.float32),
            pltpu.SemaphoreType.DMA,
        ],
    )
    out = kfn(x_num.T, xcat_flat, weight, bias, emb_table,
              offs_pat, dstc, dstn)
    return out.reshape(B, NTOK, DT)
